# baseline (speedup 1.0000x reference)
import functools

import jax
import jax.numpy as jnp
from jax import lax
from jax.experimental import pallas as pl
from jax.experimental.pallas import tpu as pltpu
from jax.experimental.pallas import tpu_sc as plsc

NC, NS = 2, 16
NW = NC * NS
CHUNK = 128


def _sc_gather_body(dims, *refs):
    n_tab = len(dims)
    tables = refs[:n_tab]
    idxs = refs[n_tab:2 * n_tab]
    outs = refs[2 * n_tab:3 * n_tab]
    idx_vs = refs[3 * n_tab:4 * n_tab]
    rows = refs[4 * n_tab:5 * n_tab]
    sem = refs[-1]

    wid = lax.axis_index("s") * NC + lax.axis_index("c")

    for t in range(n_tab):
        n_idx = idxs[t].shape[0]
        per_w = n_idx // NW
        n_chunks = per_w // CHUNK
        base = wid * per_w
        pltpu.sync_copy(idxs[t].at[pl.ds(base, per_w)], idx_vs[t])
        for cch in range(n_chunks):
            cp = pltpu.async_copy(
                tables[t].at[idx_vs[t].at[pl.ds(cch * CHUNK, CHUNK)]],
                rows[t], sem)
            cp.wait()
            pltpu.sync_copy(rows[t],
                            outs[t].at[pl.ds(base + cch * CHUNK, CHUNK)])


def _sc_gather(tables, idxs):
    dims = tuple(t.shape[1] for t in tables)
    mesh = plsc.VectorSubcoreMesh(core_axis_name="c", subcore_axis_name="s")

    out_type = tuple(
        jax.ShapeDtypeStruct((i.shape[0], t.shape[1]), jnp.float32)
        for t, i in zip(tables, idxs))
    scratch = (
        [pltpu.VMEM((i.shape[0] // NW,), jnp.int32) for i in idxs]
        + [pltpu.VMEM((CHUNK, d), jnp.float32) for d in dims]
        + [pltpu.SemaphoreType.DMA]
    )
    kern = pl.kernel(
        functools.partial(_sc_gather_body, dims),
        out_type=out_type,
        mesh=mesh,
        scratch_types=scratch,
        compiler_params=pltpu.CompilerParams(use_tc_tiling_on_sc=False),
    )
    return kern(*tables, *idxs)


def _mlp_body(g_cell, g_assay, g25, off25, g250, g5k,
              w1c, w1a, wsh, w1q, w1r, b1, w2, b2, wo, bo, out):
    acc = jnp.dot(g_cell[...], w1c[...], preferred_element_type=jnp.float32)
    acc += jnp.dot(g_assay[...], w1a[...], preferred_element_type=jnp.float32)
    acc += jnp.dot(g250[...], w1q[...], preferred_element_type=jnp.float32)
    acc += jnp.dot(g5k[...], w1r[...], preferred_element_type=jnp.float32)
    g25v = g25[...]
    off = off25[...]
    for o in range(8):
        mask = (off == o).astype(jnp.float32)
        acc += jnp.dot(g25v * mask, wsh[o],
                       preferred_element_type=jnp.float32)
    h1 = jnp.maximum(acc + b1[...], 0.0)
    h2 = jnp.maximum(
        jnp.dot(h1, w2[...], preferred_element_type=jnp.float32) + b2[...], 0.0)
    out[...] = jnp.dot(h2, wo[...], preferred_element_type=jnp.float32) + bo[...]


def _tc_mlp(gs, ws, b1, W2, b2, Wo, bo, block_b=2048):
    B = gs[0].shape[0]
    nb = B // block_b

    def full(a):
        return pl.BlockSpec(a.shape, lambda i: (0,) * a.ndim)

    in_specs = (
        [pl.BlockSpec((block_b, g.shape[1]), lambda i: (i, 0)) for g in gs]
        + [full(w) for w in ws]
        + [full(b1), full(W2), full(b2), full(Wo), full(bo)]
    )
    return pl.pallas_call(
        _mlp_body,
        grid=(nb,),
        in_specs=in_specs,
        out_specs=pl.BlockSpec((block_b, 1), lambda i: (i, 0)),
        out_shape=jax.ShapeDtypeStruct((B, 1), jnp.float32),
    )(*gs, *ws, b1, W2, b2, Wo, bo)


def kernel(x, cell_emb, assay_emb, p25_emb, p250_emb, p5k_emb,
           W1, b1, W2, b2, Wo, bo):
    x = x.astype(jnp.int32)
    B = x.shape[0]
    d25 = p25_emb.shape[1]

    i_cell, i_assay, i25, i250, i5k = (x[:, j] for j in range(5))
    g0 = (i25 * d25) >> 3
    idx4 = (g0[:, None] + jnp.arange(4, dtype=jnp.int32)).reshape(-1)
    off25 = (i25 & 7).reshape(B, 1)

    p25_view = p25_emb.reshape(-1, 8)
    p5k_pad = jnp.pad(p5k_emb, ((0, 0), (0, 3)))

    tables = (cell_emb, assay_emb, p25_view, p250_emb, p5k_pad)
    idxs = (i_cell, i_assay, idx4, i250, i5k)
    g_cell, g_assay, g25_raw, g250, g5k = _sc_gather(tables, idxs)
    g25 = g25_raw.reshape(B, 32)

    dims = [t.shape[1] for t in (cell_emb, assay_emb, p25_emb, p250_emb,
                                 p5k_emb)]
    offs = [0]
    for d in dims:
        offs.append(offs[-1] + d)
    w1c = W1[offs[0]:offs[1]]
    w1a = W1[offs[1]:offs[2]]
    w1p = W1[offs[2]:offs[3]]
    w1q = W1[offs[3]:offs[4]]
    w1r = jnp.pad(W1[offs[4]:offs[5]], ((0, 3), (0, 0)))
    wsh = jnp.stack([jnp.pad(w1p, ((o, 7 - o), (0, 0))) for o in range(8)])

    return _tc_mlp((g_cell, g_assay, g25, off25, g250, g5k),
                   (w1c, w1a, wsh, w1q, w1r),
                   b1.reshape(1, -1), W2, b2.reshape(1, -1),
                   Wo, bo.reshape(1, 1))


# speedup vs baseline: 1.3307x; 1.3307x over previous
import functools

import jax
import jax.numpy as jnp
from jax import lax
from jax.experimental import pallas as pl
from jax.experimental.pallas import tpu as pltpu
from jax.experimental.pallas import tpu_sc as plsc

NC, NS = 2, 16
NW = NC * NS
CHUNK = 128


def _repack_body(d, d_pad, x_ref, out_ref):
    x = x_ref[...]
    row = lax.broadcasted_iota(jnp.int32, (d, d_pad), 0)
    col = lax.broadcasted_iota(jnp.int32, (d, d_pad), 1)
    eye = (row == col).astype(jnp.float32)
    y = lax.dot_general(x, eye, (((0,), (0,)), ((), ())),
                        preferred_element_type=jnp.float32)
    g = 128 // d_pad
    if g > 1:
        c = y.shape[0]
        t = y.reshape(c // g, g, d_pad)
        y = jnp.concatenate([t[:, a, :] for a in range(g)], axis=-1)
    out_ref[...] = y


def _repack(table_t, d_pad, block_c):
    d, n = table_t.shape
    nb = pl.cdiv(n, block_c)
    out_rows = n * d_pad // 128
    return pl.pallas_call(
        functools.partial(_repack_body, d, d_pad),
        grid=(nb,),
        in_specs=[pl.BlockSpec((d, block_c), lambda i: (0, i))],
        out_specs=pl.BlockSpec((block_c * d_pad // 128, 128),
                               lambda i: (i, 0)),
        out_shape=jax.ShapeDtypeStruct((out_rows, 128), jnp.float32),
        compiler_params=pltpu.CompilerParams(
            dimension_semantics=("parallel",)),
    )(table_t)


_COLS = (0, 32, 64, 128, 256)


def _sc_gather_body(dims, *refs):
    n_tab = len(dims)
    tables = refs[:n_tab]
    idxs = refs[n_tab:2 * n_tab]
    out = refs[2 * n_tab]
    idx_vs = refs[2 * n_tab + 1:3 * n_tab + 1]
    rows = refs[3 * n_tab + 1:4 * n_tab + 1]
    sem = refs[-1]

    b_per_w = idx_vs[0].shape[0]
    n_chunks = b_per_w // CHUNK
    wid = lax.axis_index("s") * NC + lax.axis_index("c")
    base = wid * b_per_w

    for t in range(n_tab):
        pltpu.sync_copy(idxs[t].at[pl.ds(base, b_per_w)], idx_vs[t])
        for cch in range(n_chunks):
            cp = pltpu.async_copy(
                tables[t].at[idx_vs[t].at[pl.ds(cch * CHUNK, CHUNK)]],
                rows[t], sem)
            cp.wait()
            pltpu.sync_copy(
                rows[t],
                out.at[pl.ds(base + cch * CHUNK, CHUNK),
                       pl.ds(_COLS[t], dims[t])])


def _sc_gather(tables, idxs):
    B = idxs[0].shape[0]
    dims = tuple(t.shape[1] for t in tables)
    b_per_w = B // NW
    mesh = plsc.VectorSubcoreMesh(core_axis_name="c", subcore_axis_name="s")

    out_type = jax.ShapeDtypeStruct((B, 512), jnp.float32)
    scratch = (
        [pltpu.VMEM((b_per_w,), jnp.int32) for _ in tables]
        + [pltpu.VMEM((CHUNK, d), jnp.float32) for d in dims]
        + [pltpu.SemaphoreType.DMA]
    )
    kern = pl.kernel(
        functools.partial(_sc_gather_body, dims),
        out_type=out_type,
        mesh=mesh,
        scratch_types=scratch,
        compiler_params=pltpu.CompilerParams(use_tc_tiling_on_sc=False),
    )
    return kern(*tables, *idxs)


def _mlp_body(g, w1, b1, w2, b2, wo, bo, out):
    h1 = jnp.maximum(
        jnp.dot(g[...], w1[...], preferred_element_type=jnp.float32)
        + b1[...], 0.0)
    h2 = jnp.maximum(
        jnp.dot(h1, w2[...], preferred_element_type=jnp.float32)
        + b2[...], 0.0)
    out[...] = jnp.dot(h2, wo[...], preferred_element_type=jnp.float32) \
        + bo[...]


def _tc_mlp(g_all, W1big, b1, W2, b2, Wo, bo, block_b=2048):
    B = g_all.shape[0]
    nb = B // block_b

    def full(a):
        return pl.BlockSpec(a.shape, lambda i: (0,) * a.ndim)

    return pl.pallas_call(
        _mlp_body,
        grid=(nb,),
        in_specs=[pl.BlockSpec((block_b, g_all.shape[1]), lambda i: (i, 0)),
                  full(W1big), full(b1), full(W2), full(b2), full(Wo),
                  full(bo)],
        out_specs=pl.BlockSpec((block_b, 1), lambda i: (i, 0)),
        out_shape=jax.ShapeDtypeStruct((B, 1), jnp.float32),
        compiler_params=pltpu.CompilerParams(
            dimension_semantics=("parallel",)),
    )(g_all, W1big, b1, W2, b2, Wo, bo)


def kernel(x, cell_emb, assay_emb, p25_emb, p250_emb, p5k_emb,
           W1, b1, W2, b2, Wo, bo):
    x = x.astype(jnp.int32)
    B = x.shape[0]

    p25_r = _repack(p25_emb.T, 32, 3200)
    p250_r = _repack(p250_emb.T, 128, 2048)
    p5k_r = _repack(p5k_emb.T, 64, 2048)

    p25_v = p25_r.reshape(-1, 32)
    p5k_v = p5k_r.reshape(-1, 64)

    i_cell, i_assay, i25, i250, i5k = (x[:, j] for j in range(5))
    tables = (p25_v, cell_emb, p5k_v, p250_r, assay_emb)
    idxs = (i25, i_cell, i5k, i250, i_assay)
    g_all = _sc_gather(tables, idxs)

    w1c = W1[0:32]
    w1a = W1[32:288]
    w1p = W1[288:313]
    w1q = W1[313:353]
    w1r = W1[353:398]
    W1big = jnp.concatenate([
        jnp.pad(w1p, ((0, 7), (0, 0))),
        w1c,
        jnp.pad(w1r, ((0, 19), (0, 0))),
        jnp.pad(w1q, ((0, 88), (0, 0))),
        w1a,
    ], axis=0)

    return _tc_mlp(g_all, W1big, b1.reshape(1, -1), W2, b2.reshape(1, -1),
                   Wo, bo.reshape(1, 1))


# speedup vs baseline: 3.1449x; 2.3634x over previous
import functools

import jax
import jax.numpy as jnp
from jax import lax
from jax.experimental import pallas as pl
from jax.experimental.pallas import tpu as pltpu
from jax.experimental.pallas import tpu_sc as plsc

NC, NS = 2, 16
NW = NC * NS
CHUNK = 128


def _repack_body(d, d_pad, x_ref, out_ref):
    x = x_ref[...]
    g = 128 // d_pad
    h = x.shape[1] // g
    zpad = jnp.zeros((d_pad - d, h), jnp.float32)
    parts = []
    for q in range(g):
        parts.append(x[:, q * h:(q + 1) * h])
        parts.append(zpad)
    xs = jnp.concatenate(parts, axis=0)
    out_ref[...] = jnp.transpose(xs)


def _repack(table_t, d_pad, block_c):
    d, n = table_t.shape
    nb = pl.cdiv(n, block_c)
    g = 128 // d_pad
    out_rows = nb * (block_c // g)
    return pl.pallas_call(
        functools.partial(_repack_body, d, d_pad),
        grid=(nb,),
        in_specs=[pl.BlockSpec((d, block_c), lambda i: (0, i))],
        out_specs=pl.BlockSpec((block_c // g, 128), lambda i: (i, 0)),
        out_shape=jax.ShapeDtypeStruct((out_rows, 128), jnp.float32),
        compiler_params=pltpu.CompilerParams(
            dimension_semantics=("parallel",)),
    )(table_t)


def _perm_rows(i, block_c, d_pad):
    g = 128 // d_pad
    if g == 1:
        return i
    h = block_c // g
    c = i // block_c
    r = i % block_c
    q = r // h
    m = r % h
    return (h * c + m) * g + q


_COLS = (0, 32, 64, 128, 256)


def _sc_gather_body(dims, *refs):
    n_tab = len(dims)
    tables = refs[:n_tab]
    idxs = refs[n_tab:2 * n_tab]
    out = refs[2 * n_tab]
    idx_vs = refs[2 * n_tab + 1:3 * n_tab + 1]
    rows = refs[3 * n_tab + 1:4 * n_tab + 1]
    sem = refs[-1]

    b_per_w = idx_vs[0].shape[0]
    n_chunks = b_per_w // CHUNK
    wid = lax.axis_index("s") * NC + lax.axis_index("c")
    base = wid * b_per_w

    for t in range(n_tab):
        pltpu.sync_copy(idxs[t].at[pl.ds(base, b_per_w)], idx_vs[t])
        for cch in range(n_chunks):
            cp = pltpu.async_copy(
                tables[t].at[idx_vs[t].at[pl.ds(cch * CHUNK, CHUNK)]],
                rows[t], sem)
            cp.wait()
            pltpu.sync_copy(
                rows[t],
                out.at[pl.ds(base + cch * CHUNK, CHUNK),
                       pl.ds(_COLS[t], dims[t])])


def _sc_gather(tables, idxs):
    B = idxs[0].shape[0]
    dims = tuple(t.shape[1] for t in tables)
    b_per_w = B // NW
    mesh = plsc.VectorSubcoreMesh(core_axis_name="c", subcore_axis_name="s")

    out_type = jax.ShapeDtypeStruct((B, 512), jnp.float32)
    scratch = (
        [pltpu.VMEM((b_per_w,), jnp.int32) for _ in tables]
        + [pltpu.VMEM((CHUNK, d), jnp.float32) for d in dims]
        + [pltpu.SemaphoreType.DMA]
    )
    kern = pl.kernel(
        functools.partial(_sc_gather_body, dims),
        out_type=out_type,
        mesh=mesh,
        scratch_types=scratch,
        compiler_params=pltpu.CompilerParams(use_tc_tiling_on_sc=False),
    )
    return kern(*tables, *idxs)


def _mlp_body(g, w1, b1, w2, b2, wo, bo, out):
    h1 = jnp.maximum(
        jnp.dot(g[...], w1[...], preferred_element_type=jnp.float32)
        + b1[...], 0.0)
    h2 = jnp.maximum(
        jnp.dot(h1, w2[...], preferred_element_type=jnp.float32)
        + b2[...], 0.0)
    out[...] = jnp.dot(h2, wo[...], preferred_element_type=jnp.float32) \
        + bo[...]


def _tc_mlp(g_all, W1big, b1, W2, b2, Wo, bo, block_b=2048):
    B = g_all.shape[0]
    nb = B // block_b

    def full(a):
        return pl.BlockSpec(a.shape, lambda i: (0,) * a.ndim)

    return pl.pallas_call(
        _mlp_body,
        grid=(nb,),
        in_specs=[pl.BlockSpec((block_b, g_all.shape[1]), lambda i: (i, 0)),
                  full(W1big), full(b1), full(W2), full(b2), full(Wo),
                  full(bo)],
        out_specs=pl.BlockSpec((block_b, 1), lambda i: (i, 0)),
        out_shape=jax.ShapeDtypeStruct((B, 1), jnp.float32),
        compiler_params=pltpu.CompilerParams(
            dimension_semantics=("parallel",)),
    )(g_all, W1big, b1, W2, b2, Wo, bo)


def kernel(x, cell_emb, assay_emb, p25_emb, p250_emb, p5k_emb,
           W1, b1, W2, b2, Wo, bo):
    x = x.astype(jnp.int32)
    B = x.shape[0]

    BC = 8192
    p25_r = _repack(p25_emb.T, 32, BC)
    p250_r = _repack(p250_emb.T, 64, BC)
    p5k_r = _repack(p5k_emb.T, 128, BC)

    p25_v = p25_r.reshape(-1, 32)
    p250_v = p250_r.reshape(-1, 64)

    i_cell, i_assay, i25, i250, i5k = (x[:, j] for j in range(5))
    tables = (p25_v, cell_emb, p250_v, p5k_r, assay_emb)
    idxs = (_perm_rows(i25, BC, 32), i_cell, _perm_rows(i250, BC, 64),
            i5k, i_assay)
    g_all = _sc_gather(tables, idxs)

    w1c = W1[0:32]
    w1a = W1[32:288]
    w1p = W1[288:313]
    w1q = W1[313:353]
    w1r = W1[353:398]
    W1big = jnp.concatenate([
        jnp.pad(w1p, ((0, 7), (0, 0))),
        w1c,
        jnp.pad(w1q, ((0, 24), (0, 0))),
        jnp.pad(w1r, ((0, 83), (0, 0))),
        w1a,
    ], axis=0)

    return _tc_mlp(g_all, W1big, b1.reshape(1, -1), W2, b2.reshape(1, -1),
                   Wo, bo.reshape(1, 1))
